# baseline (speedup 1.0000x reference)
import functools

import jax
import jax.numpy as jnp
from jax import lax
from jax.experimental import pallas as pl
from jax.experimental.pallas import tpu as pltpu
from jax.experimental.pallas import tpu_sc as plsc

NUM_EMBEDDINGS = 1000000
EMBEDDING_DIM = 32

NC = 2
NS = 16
NW = NC * NS

B = 4096 * 50
B_PER_W = B // NW
CHUNK = 1600
NCHUNKS = B_PER_W // CHUNK


def _emb_kernel(table_hbm, idx_hbm, out_hbm, idx_v, rows_v, sem):
    wid = lax.axis_index("s") * NC + lax.axis_index("c")
    base = wid * B_PER_W

    def step(i, _):
        off = base + i * CHUNK
        pltpu.sync_copy(idx_hbm.at[pl.ds(off, CHUNK)], idx_v)
        pltpu.async_copy(table_hbm.at[idx_v], rows_v, sem).wait()
        pltpu.sync_copy(rows_v, out_hbm.at[pl.ds(off, CHUNK)])
        return ()

    lax.fori_loop(0, NCHUNKS, step, ())


@jax.jit
def _emb_lookup(idx_flat, weight):
    mesh = plsc.VectorSubcoreMesh(core_axis_name="c", subcore_axis_name="s")
    run = functools.partial(
        pl.kernel,
        mesh=mesh,
        out_type=jax.ShapeDtypeStruct((B, EMBEDDING_DIM), jnp.float32),
        scratch_types=[
            pltpu.VMEM((CHUNK,), jnp.int32),
            pltpu.VMEM((CHUNK, EMBEDDING_DIM), jnp.float32),
            pltpu.SemaphoreType.DMA,
        ],
        compiler_params=pltpu.CompilerParams(use_tc_tiling_on_sc=False),
    )(_emb_kernel)
    return run(weight, idx_flat)


def kernel(x, weight):
    idx_flat = x.reshape(-1).astype(jnp.int32)
    out = _emb_lookup(idx_flat, weight)
    return out.reshape(x.shape + (EMBEDDING_DIM,))


# speedup vs baseline: 1.0029x; 1.0029x over previous
import functools

import jax
import jax.numpy as jnp
from jax import lax
from jax.experimental import pallas as pl
from jax.experimental.pallas import tpu as pltpu
from jax.experimental.pallas import tpu_sc as plsc

NUM_EMBEDDINGS = 1000000
EMBEDDING_DIM = 32

NC = 2
NS = 16
NW = NC * NS

B = 4096 * 50
B_PER_W = B // NW
CHUNK = 800
NCHUNKS = B_PER_W // CHUNK
NBUF = 4
LOOKAHEAD = 2


def _emb_kernel(table_hbm, idx_hbm, out_hbm, idx_v, rows_v, gsems, ssems):
    wid = lax.axis_index("s") * NC + lax.axis_index("c")
    base = wid * B_PER_W

    pltpu.sync_copy(idx_hbm.at[pl.ds(base, B_PER_W)], idx_v)

    def start_gather(j):
        b = j % NBUF
        return pltpu.async_copy(
            table_hbm.at[idx_v.at[pl.ds(j * CHUNK, CHUNK)]],
            rows_v.at[b],
            gsems.at[b],
        )

    def start_store(j):
        b = j % NBUF
        return pltpu.async_copy(
            rows_v.at[b],
            out_hbm.at[pl.ds(base + j * CHUNK, CHUNK)],
            ssems.at[b],
        )

    gathers = {}
    stores = {}
    for j in range(min(LOOKAHEAD, NCHUNKS)):
        gathers[j] = start_gather(j)
    for i in range(NCHUNKS):
        gathers[i].wait()
        stores[i] = start_store(i)
        nxt = i + LOOKAHEAD
        if nxt < NCHUNKS:
            if nxt >= NBUF:
                stores[nxt - NBUF].wait()
            gathers[nxt] = start_gather(nxt)
    for i in range(max(0, NCHUNKS - NBUF), NCHUNKS):
        if i in stores:
            stores[i].wait()


@jax.jit
def _emb_lookup(idx_flat, weight):
    mesh = plsc.VectorSubcoreMesh(core_axis_name="c", subcore_axis_name="s")
    run = functools.partial(
        pl.kernel,
        mesh=mesh,
        out_type=jax.ShapeDtypeStruct((B, EMBEDDING_DIM), jnp.float32),
        scratch_types=[
            pltpu.VMEM((B_PER_W,), jnp.int32),
            pltpu.VMEM((NBUF, CHUNK, EMBEDDING_DIM), jnp.float32),
            pltpu.SemaphoreType.DMA((NBUF,)),
            pltpu.SemaphoreType.DMA((NBUF,)),
        ],
        compiler_params=pltpu.CompilerParams(use_tc_tiling_on_sc=False),
    )(_emb_kernel)
    return run(weight, idx_flat)


def kernel(x, weight):
    idx_flat = x.reshape(-1).astype(jnp.int32)
    out = _emb_lookup(idx_flat, weight)
    return out.reshape(x.shape + (EMBEDDING_DIM,))
